# baseline (speedup 1.0000x reference)
import functools

import jax
import jax.numpy as jnp
from jax import lax
from jax.experimental import pallas as pl
from jax.experimental.pallas import tpu as pltpu
from jax.experimental.pallas import tpu_sc as plsc

_VOCAB = 100000
_MAX_LEN = 200
_EMBED_DIM = 100
_NUM_CLASSES = 5
_BATCH = 4096

_NC = 2
_NS = 16
_NW = _NC * _NS
_B_PER_TILE = _BATCH // _NW
_B_BLK = 4
_N_GROUPS = _B_PER_TILE // _B_BLK
_TOT_GROUPS = _BATCH // _B_BLK
_CHL = 50
_NCH = _MAX_LEN // _CHL
_GR = _B_BLK * _CHL
_DPAD = 112
_KV = _DPAD // 16
_NGRP = 4


def _sc_body(table, bidx, wpk, out, w_v, rows_v, idx_v, out_v,
             sem_rows, sem_idx):
    cid = lax.axis_index("c")
    sid = lax.axis_index("s")
    wid = sid * _NC + cid
    g0 = wid * _N_GROUPS

    pltpu.sync_copy(wpk, w_v)

    pltpu.sync_copy(bidx.at[g0], idx_v.at[0])
    pltpu.async_copy(table.at[idx_v.at[0, 0]], rows_v.at[0], sem_rows)
    pltpu.async_copy(bidx.at[g0 + 1], idx_v.at[1], sem_idx)

    def wait_rows(p):
        pltpu.make_async_copy(
            table.at[pl.ds(0, _GR)], rows_v.at[p], sem_rows).wait()

    def wait_idx():
        pltpu.make_async_copy(bidx.at[0], idx_v.at[0], sem_idx).wait()

    def compute_chunk(p, ci, acc):
        @plsc.parallel_loop(0, _CHL, unroll=2, carry=tuple(acc))
        def acc_out(l, acc):
            acc = list(acc)
            lg = ci * _CHL + l
            for kh in (range(0, 4), range(4, _KV)):
                xs = [
                    [rows_v[p, j * _CHL + l, pl.ds(k * 16, 16)] for k in kh]
                    for j in range(_B_BLK)
                ]
                for c in range(_NUM_CLASSES):
                    ws = []
                    unpacked = {}
                    for k in kh:
                        gq, half = divmod(k, 2)
                        if gq not in unpacked:
                            wword = w_v[c, lg, gq, :]
                            unpacked[gq] = plsc.unpack(
                                wword,
                                format=plsc.PackFormat.INTERLEAVED,
                                preferred_element_type=jnp.float32,
                            )
                        ws.append(unpacked[gq][half])
                    for j in range(_B_BLK):
                        for i, _ in enumerate(kh):
                            acc[j * _NUM_CLASSES + c] = (
                                acc[j * _NUM_CLASSES + c] + xs[j][i] * ws[i]
                            )
            return tuple(acc)

        return list(acc_out)

    def group_body(g, carry):
        gi = g % 4
        acc = [jnp.zeros((16,), jnp.float32)] * (_B_BLK * _NUM_CLASSES)
        for ci in range(_NCH):
            p = ci % 2
            wait_rows(p)
            if ci < _NCH - 1:
                pltpu.async_copy(
                    table.at[idx_v.at[gi, ci + 1]], rows_v.at[1 - p],
                    sem_rows)
            else:
                @pl.when(g < _N_GROUPS - 1)
                def _():
                    wait_idx()
                    pltpu.async_copy(
                        table.at[idx_v.at[(g + 1) % 4, 0]], rows_v.at[1 - p],
                        sem_rows)

                @pl.when(g < _N_GROUPS - 2)
                def _():
                    pltpu.async_copy(
                        bidx.at[g0 + g + 2], idx_v.at[(g + 2) % 4], sem_idx)
            acc = compute_chunk(p, ci, acc)

        lane = lax.iota(jnp.int32, 16)
        for j in range(_B_BLK):
            svec = jnp.zeros((16,), jnp.float32)
            for c in range(_NUM_CLASSES):
                s = jnp.broadcast_to(jnp.sum(acc[j * _NUM_CLASSES + c]), (16,))
                svec = jnp.where(lane == c, s, svec)
            out_v[g * _B_BLK + j, :] = svec
        return carry

    lax.fori_loop(0, _N_GROUPS, group_body, 0)
    pltpu.sync_copy(out_v, out.at[pl.ds(wid * _B_PER_TILE, _B_PER_TILE)])


def _sc_logits(table_pad, bidx, w_packed):
    mesh = plsc.VectorSubcoreMesh(
        core_axis_name="c", subcore_axis_name="s",
        num_cores=_NC, num_subcores=_NS,
    )
    call = functools.partial(
        pl.kernel,
        out_type=jax.ShapeDtypeStruct((_BATCH, 16), jnp.float32),
        mesh=mesh,
        scratch_types=[
            pltpu.VMEM((_NUM_CLASSES, _MAX_LEN, _NGRP, 32), jnp.bfloat16),
            pltpu.VMEM((2, _GR, _DPAD), jnp.float32),
            pltpu.VMEM((4, _NCH, _GR), jnp.int32),
            pltpu.VMEM((_B_PER_TILE, 16), jnp.float32),
            pltpu.SemaphoreType.DMA,
            pltpu.SemaphoreType.DMA,
        ],
        compiler_params=pltpu.CompilerParams(
            needs_layout_passes=False, use_tc_tiling_on_sc=False),
    )(_sc_body)
    return call(table_pad, bidx, w_packed)


def _tc_logsoftmax_body(x_ref, b_ref, o_ref):
    x = x_ref[...][:, : _NUM_CLASSES] + b_ref[...]
    m = jnp.max(x, axis=-1, keepdims=True)
    e = jnp.exp(x - m)
    o_ref[...] = (x - m) - jnp.log(jnp.sum(e, axis=-1, keepdims=True))


def _pack_weights(fc_w):
    w3 = fc_w.reshape(_NUM_CLASSES, _MAX_LEN, _EMBED_DIM)
    w_pad = jnp.pad(w3, ((0, 0), (0, 0), (0, _DPAD - _EMBED_DIM)))
    wr = w_pad.reshape(_NUM_CLASSES, _MAX_LEN, _KV, 16)
    zero = jnp.zeros_like(wr[:, :, 0])
    a_half = jnp.stack([wr[:, :, 0], wr[:, :, 2], wr[:, :, 4], wr[:, :, 6]], axis=2)
    b_half = jnp.stack([wr[:, :, 1], wr[:, :, 3], wr[:, :, 5], zero], axis=2)
    inter = jnp.stack([a_half, b_half], axis=-1)
    return inter.reshape(_NUM_CLASSES, _MAX_LEN, _NGRP, 32).astype(jnp.bfloat16)


def kernel(batch, emb_table, fc_w, fc_b):
    table_pad = jnp.pad(emb_table, ((0, 0), (0, _DPAD - _EMBED_DIM)))
    bidx = (
        batch.astype(jnp.int32)
        .reshape(_TOT_GROUPS, _B_BLK, _NCH, _CHL)
        .transpose(0, 2, 1, 3)
        .reshape(_TOT_GROUPS, _NCH, _GR)
    )
    w_packed = _pack_weights(fc_w)

    logits = _sc_logits(table_pad, bidx, w_packed)

    return pl.pallas_call(
        _tc_logsoftmax_body,
        out_shape=jax.ShapeDtypeStruct((_BATCH, _NUM_CLASSES), jnp.float32),
    )(logits, fc_b.reshape(1, _NUM_CLASSES))


# speedup vs baseline: 1.3153x; 1.3153x over previous
import functools

import jax
import jax.numpy as jnp
from jax import lax
from jax.experimental import pallas as pl
from jax.experimental.pallas import tpu as pltpu
from jax.experimental.pallas import tpu_sc as plsc

_VOCAB = 100000
_MAX_LEN = 200
_EMBED_DIM = 100
_NUM_CLASSES = 5
_BATCH = 4096

_NC = 2
_NS = 16
_NW = _NC * _NS
_B_PER_TILE = _BATCH // _NW
_B_BLK = 4
_N_GROUPS = _B_PER_TILE // _B_BLK
_TOT_GROUPS = _BATCH // _B_BLK
_CHL = 50
_NCH = _MAX_LEN // _CHL
_GR = _B_BLK * _CHL
_DPAD = 112
_KV = _DPAD // 16
_NGRP = 4


def _sc_body(table, bidx, wpk, out, w_v, rows_v, idx_v, out_v, acc_v,
             sem_rows, sem_idx):
    cid = lax.axis_index("c")
    sid = lax.axis_index("s")
    wid = sid * _NC + cid
    g0 = wid * _N_GROUPS

    pltpu.sync_copy(wpk, w_v)

    pltpu.sync_copy(bidx.at[g0], idx_v.at[0])
    pltpu.async_copy(table.at[idx_v.at[0, 0]], rows_v.at[0], sem_rows)
    pltpu.async_copy(bidx.at[g0 + 1], idx_v.at[1], sem_idx)

    def wait_rows(p):
        pltpu.make_async_copy(
            table.at[pl.ds(0, _GR)], rows_v.at[p], sem_rows).wait()

    def wait_idx():
        pltpu.make_async_copy(bidx.at[0], idx_v.at[0], sem_idx).wait()

    def compute_chunk(p, ci):
        for kp in range(_NGRP):
            def l_body(l, carry, kp=kp):
                lg = ci * _CHL + l
                wev, wod = [], []
                for c in range(_NUM_CLASSES):
                    up = plsc.unpack(
                        w_v[c, lg, kp, :],
                        format=plsc.PackFormat.INTERLEAVED,
                        preferred_element_type=jnp.float32,
                    )
                    wev.append(up[0])
                    if kp < _NGRP - 1:
                        wod.append(up[1])
                for j in range(_B_BLK):
                    xe = rows_v[p, j * _CHL + l, pl.ds(kp * 32, 16)]
                    if kp < _NGRP - 1:
                        xo = rows_v[p, j * _CHL + l, pl.ds(kp * 32 + 16, 16)]
                    for c in range(_NUM_CLASSES):
                        if kp < _NGRP - 1:
                            contrib = xe * wev[c] + xo * wod[c]
                        else:
                            contrib = xe * wev[c]
                        plsc.addupdate(acc_v.at[j, c], contrib)
                return carry

            lax.fori_loop(0, _CHL, l_body, 0)

    def group_body(g, carry):
        gi = g % 4
        zero16 = jnp.zeros((16,), jnp.float32)
        for j in range(_B_BLK):
            for c in range(_NUM_CLASSES):
                acc_v[j, c, :] = zero16
        for ci in range(_NCH):
            p = ci % 2
            wait_rows(p)
            if ci < _NCH - 1:
                pltpu.async_copy(
                    table.at[idx_v.at[gi, ci + 1]], rows_v.at[1 - p],
                    sem_rows)
            else:
                @pl.when(g < _N_GROUPS - 1)
                def _():
                    wait_idx()
                    pltpu.async_copy(
                        table.at[idx_v.at[(g + 1) % 4, 0]], rows_v.at[1 - p],
                        sem_rows)

                @pl.when(g < _N_GROUPS - 2)
                def _():
                    pltpu.async_copy(
                        bidx.at[g0 + g + 2], idx_v.at[(g + 2) % 4], sem_idx)
            compute_chunk(p, ci)

        lane = lax.iota(jnp.int32, 16)
        for j in range(_B_BLK):
            svec = jnp.zeros((16,), jnp.float32)
            for c in range(_NUM_CLASSES):
                s = jnp.broadcast_to(jnp.sum(acc_v[j, c, :]), (16,))
                svec = jnp.where(lane == c, s, svec)
            out_v[g * _B_BLK + j, :] = svec
        return carry

    lax.fori_loop(0, _N_GROUPS, group_body, 0)
    pltpu.sync_copy(out_v, out.at[pl.ds(wid * _B_PER_TILE, _B_PER_TILE)])


def _sc_logits(table_pad, bidx, w_packed):
    mesh = plsc.VectorSubcoreMesh(
        core_axis_name="c", subcore_axis_name="s",
        num_cores=_NC, num_subcores=_NS,
    )
    call = functools.partial(
        pl.kernel,
        out_type=jax.ShapeDtypeStruct((_BATCH, 16), jnp.float32),
        mesh=mesh,
        scratch_types=[
            pltpu.VMEM((_NUM_CLASSES, _MAX_LEN, _NGRP, 32), jnp.bfloat16),
            pltpu.VMEM((2, _GR, _DPAD), jnp.float32),
            pltpu.VMEM((4, _NCH, _GR), jnp.int32),
            pltpu.VMEM((_B_PER_TILE, 16), jnp.float32),
            pltpu.VMEM((_B_BLK, _NUM_CLASSES, 16), jnp.float32),
            pltpu.SemaphoreType.DMA,
            pltpu.SemaphoreType.DMA,
        ],
        compiler_params=pltpu.CompilerParams(
            needs_layout_passes=False, use_tc_tiling_on_sc=False),
    )(_sc_body)
    return call(table_pad, bidx, w_packed)


def _tc_logsoftmax_body(x_ref, b_ref, o_ref):
    x = x_ref[...][:, : _NUM_CLASSES] + b_ref[...]
    m = jnp.max(x, axis=-1, keepdims=True)
    e = jnp.exp(x - m)
    o_ref[...] = (x - m) - jnp.log(jnp.sum(e, axis=-1, keepdims=True))


def _pack_weights(fc_w):
    w3 = fc_w.reshape(_NUM_CLASSES, _MAX_LEN, _EMBED_DIM)
    w_pad = jnp.pad(w3, ((0, 0), (0, 0), (0, _DPAD - _EMBED_DIM)))
    wr = w_pad.reshape(_NUM_CLASSES, _MAX_LEN, _KV, 16)
    zero = jnp.zeros_like(wr[:, :, 0])
    a_half = jnp.stack([wr[:, :, 0], wr[:, :, 2], wr[:, :, 4], wr[:, :, 6]], axis=2)
    b_half = jnp.stack([wr[:, :, 1], wr[:, :, 3], wr[:, :, 5], zero], axis=2)
    inter = jnp.stack([a_half, b_half], axis=-1)
    return inter.reshape(_NUM_CLASSES, _MAX_LEN, _NGRP, 32).astype(jnp.bfloat16)


def kernel(batch, emb_table, fc_w, fc_b):
    table_pad = jnp.pad(emb_table, ((0, 0), (0, _DPAD - _EMBED_DIM)))
    bidx = (
        batch.astype(jnp.int32)
        .reshape(_TOT_GROUPS, _B_BLK, _NCH, _CHL)
        .transpose(0, 2, 1, 3)
        .reshape(_TOT_GROUPS, _NCH, _GR)
    )
    w_packed = _pack_weights(fc_w)

    logits = _sc_logits(table_pad, bidx, w_packed)

    return pl.pallas_call(
        _tc_logsoftmax_body,
        out_shape=jax.ShapeDtypeStruct((_BATCH, _NUM_CLASSES), jnp.float32),
    )(logits, fc_b.reshape(1, _NUM_CLASSES))


# speedup vs baseline: 1.7362x; 1.3200x over previous
import functools

import jax
import jax.numpy as jnp
from jax import lax
from jax.experimental import pallas as pl
from jax.experimental.pallas import tpu as pltpu
from jax.experimental.pallas import tpu_sc as plsc

_VOCAB = 100000
_MAX_LEN = 200
_EMBED_DIM = 100
_NUM_CLASSES = 5
_BATCH = 4096

_NC = 2
_NS = 16
_NW = _NC * _NS
_B_PER_TILE = _BATCH // _NW
_B_BLK = 4
_N_GROUPS = _B_PER_TILE // _B_BLK
_TOT_GROUPS = _BATCH // _B_BLK
_CHL = 50
_NCH = _MAX_LEN // _CHL
_GR = _B_BLK * _CHL
_DPAD = 112
_KV = _DPAD // 16
_NGRP = 4
_GRP_SPLIT = (
    ((0, (0, 16)), (1, (32, 48))),
    ((2, (64, 80)), (3, (96, None))),
)


def _sc_body(table, bidx, wpk, out, w_v, rows_v, idx_v, out_v, acc_v,
             sem_rows, sem_idx):
    cid = lax.axis_index("c")
    sid = lax.axis_index("s")
    wid = sid * _NC + cid
    g0 = wid * _N_GROUPS

    pltpu.sync_copy(wpk, w_v)

    pltpu.sync_copy(bidx.at[g0], idx_v.at[0])
    pltpu.async_copy(table.at[idx_v.at[0, 0]], rows_v.at[0], sem_rows)
    pltpu.async_copy(bidx.at[g0 + 1], idx_v.at[1], sem_idx)

    def wait_rows(p):
        pltpu.make_async_copy(
            table.at[pl.ds(0, _GR)], rows_v.at[p], sem_rows).wait()

    def wait_idx():
        pltpu.make_async_copy(bidx.at[0], idx_v.at[0], sem_idx).wait()

    def compute_chunk(p, ci):
        for grp in _GRP_SPLIT:
            def l_body(l, carry, grp=grp):
                lg = ci * _CHL + l
                halves = []
                for kp, offs in grp:
                    ups = [
                        plsc.unpack(
                            w_v[c, lg, kp, :],
                            format=plsc.PackFormat.INTERLEAVED,
                            preferred_element_type=jnp.float32,
                        )
                        for c in range(_NUM_CLASSES)
                    ]
                    halves.append((offs[0], [u[0] for u in ups]))
                    if offs[1] is not None:
                        halves.append((offs[1], [u[1] for u in ups]))
                for j in range(_B_BLK):
                    xs = [
                        (rows_v[p, j * _CHL + l, pl.ds(off, 16)], wl)
                        for off, wl in halves
                    ]
                    for c in range(_NUM_CLASSES):
                        terms = [x * wl[c] for x, wl in xs]
                        s = terms[0]
                        for t in terms[1:]:
                            s = s + t
                        plsc.addupdate(acc_v.at[j, c], s)
                return carry

            lax.fori_loop(0, _CHL, l_body, 0)

    def group_body(g, carry):
        gi = g % 4
        zero16 = jnp.zeros((16,), jnp.float32)
        for j in range(_B_BLK):
            for c in range(_NUM_CLASSES):
                acc_v[j, c, :] = zero16
        for ci in range(_NCH):
            p = ci % 2
            wait_rows(p)
            if ci < _NCH - 1:
                pltpu.async_copy(
                    table.at[idx_v.at[gi, ci + 1]], rows_v.at[1 - p],
                    sem_rows)
            else:
                @pl.when(g < _N_GROUPS - 1)
                def _():
                    wait_idx()
                    pltpu.async_copy(
                        table.at[idx_v.at[(g + 1) % 4, 0]], rows_v.at[1 - p],
                        sem_rows)

                @pl.when(g < _N_GROUPS - 2)
                def _():
                    pltpu.async_copy(
                        bidx.at[g0 + g + 2], idx_v.at[(g + 2) % 4], sem_idx)
            compute_chunk(p, ci)

        lane = lax.iota(jnp.int32, 16)
        for j in range(_B_BLK):
            svec = jnp.zeros((16,), jnp.float32)
            for c in range(_NUM_CLASSES):
                s = jnp.broadcast_to(jnp.sum(acc_v[j, c, :]), (16,))
                svec = jnp.where(lane == c, s, svec)
            out_v[g * _B_BLK + j, :] = svec
        return carry

    lax.fori_loop(0, _N_GROUPS, group_body, 0)
    pltpu.sync_copy(out_v, out.at[pl.ds(wid * _B_PER_TILE, _B_PER_TILE)])


def _sc_logits(table_pad, bidx, w_packed):
    mesh = plsc.VectorSubcoreMesh(
        core_axis_name="c", subcore_axis_name="s",
        num_cores=_NC, num_subcores=_NS,
    )
    call = functools.partial(
        pl.kernel,
        out_type=jax.ShapeDtypeStruct((_BATCH, 16), jnp.float32),
        mesh=mesh,
        scratch_types=[
            pltpu.VMEM((_NUM_CLASSES, _MAX_LEN, _NGRP, 32), jnp.bfloat16),
            pltpu.VMEM((2, _GR, _DPAD), jnp.float32),
            pltpu.VMEM((4, _NCH, _GR), jnp.int32),
            pltpu.VMEM((_B_PER_TILE, 16), jnp.float32),
            pltpu.VMEM((_B_BLK, _NUM_CLASSES, 16), jnp.float32),
            pltpu.SemaphoreType.DMA,
            pltpu.SemaphoreType.DMA,
        ],
        compiler_params=pltpu.CompilerParams(
            needs_layout_passes=False, use_tc_tiling_on_sc=False),
    )(_sc_body)
    return call(table_pad, bidx, w_packed)


def _tc_logsoftmax_body(x_ref, b_ref, o_ref):
    x = x_ref[...][:, : _NUM_CLASSES] + b_ref[...]
    m = jnp.max(x, axis=-1, keepdims=True)
    e = jnp.exp(x - m)
    o_ref[...] = (x - m) - jnp.log(jnp.sum(e, axis=-1, keepdims=True))


def _pack_weights(fc_w):
    w3 = fc_w.reshape(_NUM_CLASSES, _MAX_LEN, _EMBED_DIM)
    w_pad = jnp.pad(w3, ((0, 0), (0, 0), (0, _DPAD - _EMBED_DIM)))
    wr = w_pad.reshape(_NUM_CLASSES, _MAX_LEN, _KV, 16)
    zero = jnp.zeros_like(wr[:, :, 0])
    a_half = jnp.stack([wr[:, :, 0], wr[:, :, 2], wr[:, :, 4], wr[:, :, 6]], axis=2)
    b_half = jnp.stack([wr[:, :, 1], wr[:, :, 3], wr[:, :, 5], zero], axis=2)
    inter = jnp.stack([a_half, b_half], axis=-1)
    return inter.reshape(_NUM_CLASSES, _MAX_LEN, _NGRP, 32).astype(jnp.bfloat16)


def _tc_pad_body(x_ref, o_ref):
    o_ref[...] = jnp.pad(x_ref[...], ((0, 0), (0, _DPAD - _EMBED_DIM)))


def _pad_table(emb_table):
    rows_blk = _VOCAB // 10
    return pl.pallas_call(
        _tc_pad_body,
        grid=(10,),
        in_specs=[pl.BlockSpec((rows_blk, _EMBED_DIM), lambda i: (i, 0))],
        out_specs=pl.BlockSpec((rows_blk, _DPAD), lambda i: (i, 0)),
        out_shape=jax.ShapeDtypeStruct((_VOCAB, _DPAD), jnp.float32),
    )(emb_table)


def kernel(batch, emb_table, fc_w, fc_b):
    bidx = (
        batch.astype(jnp.int32)
        .reshape(_TOT_GROUPS, _B_BLK, _NCH, _CHL)
        .transpose(0, 2, 1, 3)
        .reshape(_TOT_GROUPS, _NCH, _GR)
    )
    w_packed = _pack_weights(fc_w)

    logits = _sc_logits(_pad_table(emb_table), bidx, w_packed)

    return pl.pallas_call(
        _tc_logsoftmax_body,
        out_shape=jax.ShapeDtypeStruct((_BATCH, _NUM_CLASSES), jnp.float32),
    )(logits, fc_b.reshape(1, _NUM_CLASSES))


# speedup vs baseline: 2.2982x; 1.3236x over previous
import functools

import jax
import jax.numpy as jnp
from jax import lax
from jax.experimental import pallas as pl
from jax.experimental.pallas import tpu as pltpu
from jax.experimental.pallas import tpu_sc as plsc

_VOCAB = 100000
_MAX_LEN = 200
_EMBED_DIM = 100
_NUM_CLASSES = 5
_BATCH = 4096

_NC = 2
_NS = 16
_NW = _NC * _NS
_B_PER_TILE = _BATCH // _NW
_B_BLK = 4
_N_GROUPS = _B_PER_TILE // _B_BLK
_TOT_GROUPS = _BATCH // _B_BLK
_CHL = 50
_NCH = _MAX_LEN // _CHL
_GR = _B_BLK * _CHL
_DPAD = 112
_KV = _DPAD // 16
_NGRP = 4
_GRP_SPLIT = (
    ((0, (0, 16)), (1, (32, 48))),
    ((2, (64, 80)), (3, (96, None))),
)


def _sc_body(table, bidx, wpk, out, w_v, rows_v, idx_v, out_v, acc_v,
             sem_rows, sem_idx):
    cid = lax.axis_index("c")
    sid = lax.axis_index("s")
    wid = sid * _NC + cid
    g0 = wid * _N_GROUPS

    pltpu.sync_copy(wpk, w_v)

    pltpu.sync_copy(bidx.at[g0], idx_v.at[0])
    pltpu.async_copy(table.at[idx_v.at[0, 0]], rows_v.at[0], sem_rows)
    pltpu.async_copy(bidx.at[g0 + 1], idx_v.at[1], sem_idx)

    def wait_rows(p):
        pltpu.make_async_copy(
            table.at[pl.ds(0, _GR)], rows_v.at[p], sem_rows).wait()

    def wait_idx():
        pltpu.make_async_copy(bidx.at[0], idx_v.at[0], sem_idx).wait()

    def compute_chunk(p, ci):
        for grp in _GRP_SPLIT:
            @plsc.parallel_loop(0, _CHL)
            def l_body(l, grp=grp):
                lg = ci * _CHL + l
                halves = []
                for kp, offs in grp:
                    ups = [
                        plsc.unpack(
                            w_v[c, lg, kp, :],
                            format=plsc.PackFormat.INTERLEAVED,
                            preferred_element_type=jnp.float32,
                        )
                        for c in range(_NUM_CLASSES)
                    ]
                    halves.append((offs[0], [u[0] for u in ups]))
                    if offs[1] is not None:
                        halves.append((offs[1], [u[1] for u in ups]))
                for j in range(_B_BLK):
                    xs = [
                        (rows_v[p, j * _CHL + l, pl.ds(off, 16)], wl)
                        for off, wl in halves
                    ]
                    for c in range(_NUM_CLASSES):
                        terms = [x * wl[c] for x, wl in xs]
                        s = terms[0]
                        for t in terms[1:]:
                            s = s + t
                        plsc.addupdate(acc_v.at[j, c], s)

    def group_body(g, carry):
        gi = g % 4
        zero16 = jnp.zeros((16,), jnp.float32)
        for j in range(_B_BLK):
            for c in range(_NUM_CLASSES):
                acc_v[j, c, :] = zero16
        for ci in range(_NCH):
            p = ci % 2
            wait_rows(p)
            if ci < _NCH - 1:
                pltpu.async_copy(
                    table.at[idx_v.at[gi, ci + 1]], rows_v.at[1 - p],
                    sem_rows)
            else:
                @pl.when(g < _N_GROUPS - 1)
                def _():
                    wait_idx()
                    pltpu.async_copy(
                        table.at[idx_v.at[(g + 1) % 4, 0]], rows_v.at[1 - p],
                        sem_rows)

                @pl.when(g < _N_GROUPS - 2)
                def _():
                    pltpu.async_copy(
                        bidx.at[g0 + g + 2], idx_v.at[(g + 2) % 4], sem_idx)
            compute_chunk(p, ci)

        lane = lax.iota(jnp.int32, 16)
        for j in range(_B_BLK):
            svec = jnp.zeros((16,), jnp.float32)
            for c in range(_NUM_CLASSES):
                s = jnp.broadcast_to(jnp.sum(acc_v[j, c, :]), (16,))
                svec = jnp.where(lane == c, s, svec)
            out_v[g * _B_BLK + j, :] = svec
        return carry

    lax.fori_loop(0, _N_GROUPS, group_body, 0)
    pltpu.sync_copy(out_v, out.at[pl.ds(wid * _B_PER_TILE, _B_PER_TILE)])


def _sc_logits(table_pad, bidx, w_packed):
    mesh = plsc.VectorSubcoreMesh(
        core_axis_name="c", subcore_axis_name="s",
        num_cores=_NC, num_subcores=_NS,
    )
    call = functools.partial(
        pl.kernel,
        out_type=jax.ShapeDtypeStruct((_BATCH, 16), jnp.float32),
        mesh=mesh,
        scratch_types=[
            pltpu.VMEM((_NUM_CLASSES, _MAX_LEN, _NGRP, 32), jnp.bfloat16),
            pltpu.VMEM((2, _GR, _DPAD), jnp.float32),
            pltpu.VMEM((4, _NCH, _GR), jnp.int32),
            pltpu.VMEM((_B_PER_TILE, 16), jnp.float32),
            pltpu.VMEM((_B_BLK, _NUM_CLASSES, 16), jnp.float32),
            pltpu.SemaphoreType.DMA,
            pltpu.SemaphoreType.DMA,
        ],
        compiler_params=pltpu.CompilerParams(
            needs_layout_passes=False, use_tc_tiling_on_sc=False),
    )(_sc_body)
    return call(table_pad, bidx, w_packed)


def _tc_logsoftmax_body(x_ref, b_ref, o_ref):
    x = x_ref[...][:, : _NUM_CLASSES] + b_ref[...]
    m = jnp.max(x, axis=-1, keepdims=True)
    e = jnp.exp(x - m)
    o_ref[...] = (x - m) - jnp.log(jnp.sum(e, axis=-1, keepdims=True))


def _pack_weights(fc_w):
    w3 = fc_w.reshape(_NUM_CLASSES, _MAX_LEN, _EMBED_DIM)
    w_pad = jnp.pad(w3, ((0, 0), (0, 0), (0, _DPAD - _EMBED_DIM)))
    wr = w_pad.reshape(_NUM_CLASSES, _MAX_LEN, _KV, 16)
    zero = jnp.zeros_like(wr[:, :, 0])
    a_half = jnp.stack([wr[:, :, 0], wr[:, :, 2], wr[:, :, 4], wr[:, :, 6]], axis=2)
    b_half = jnp.stack([wr[:, :, 1], wr[:, :, 3], wr[:, :, 5], zero], axis=2)
    inter = jnp.stack([a_half, b_half], axis=-1)
    return inter.reshape(_NUM_CLASSES, _MAX_LEN, _NGRP, 32).astype(jnp.bfloat16)


def _tc_pad_body(x_ref, o_ref):
    o_ref[...] = jnp.pad(x_ref[...], ((0, 0), (0, _DPAD - _EMBED_DIM)))


def _pad_table(emb_table):
    rows_blk = _VOCAB // 10
    return pl.pallas_call(
        _tc_pad_body,
        grid=(10,),
        in_specs=[pl.BlockSpec((rows_blk, _EMBED_DIM), lambda i: (i, 0))],
        out_specs=pl.BlockSpec((rows_blk, _DPAD), lambda i: (i, 0)),
        out_shape=jax.ShapeDtypeStruct((_VOCAB, _DPAD), jnp.float32),
    )(emb_table)


def kernel(batch, emb_table, fc_w, fc_b):
    bidx = (
        batch.astype(jnp.int32)
        .reshape(_TOT_GROUPS, _B_BLK, _NCH, _CHL)
        .transpose(0, 2, 1, 3)
        .reshape(_TOT_GROUPS, _NCH, _GR)
    )
    w_packed = _pack_weights(fc_w)

    logits = _sc_logits(_pad_table(emb_table), bidx, w_packed)

    return pl.pallas_call(
        _tc_logsoftmax_body,
        out_shape=jax.ShapeDtypeStruct((_BATCH, _NUM_CLASSES), jnp.float32),
    )(logits, fc_b.reshape(1, _NUM_CLASSES))
